# Initial kernel scaffold; baseline (speedup 1.0000x reference)
import functools

import jax
import jax.numpy as jnp
from jax import lax
from jax.experimental import pallas as pl
from jax.experimental.pallas import tpu as pltpu
from jax.experimental.pallas import tpu_sc as plsc

N = 10000
D = 256
E = 80000

NC = 2
NS = 16
HALF = D // 2
AUGC = HALF + 16
ROWS_PAD = 10240
EDGES_PER_TILE = 5120
E_PAD = EDGES_PER_TILE * NS
CHUNK = 128
N_CHUNKS = EDGES_PER_TILE // CHUNK
STRIPE = ROWS_PAD // NS


def _sc_segment_sums(aug_flat, edges):
    mesh = plsc.VectorSubcoreMesh(core_axis_name="c", subcore_axis_name="s")

    @functools.partial(
        pl.kernel,
        mesh=mesh,
        out_type=jax.ShapeDtypeStruct((2, NC, ROWS_PAD, AUGC), jnp.float32),
        scratch_types=[
            pltpu.VMEM((2, CHUNK), jnp.int32),
            pltpu.VMEM((CHUNK, AUGC), jnp.float32),
            pltpu.VMEM((16, AUGC), jnp.float32),
            pltpu.VMEM_SHARED((ROWS_PAD, AUGC), jnp.float32),
            pltpu.SemaphoreType.DMA,
        ],
    )
    def k(aug_hbm, edges_hbm, out_hbm, idx_v, rows_v, zbuf, acc, sem):
        c = lax.axis_index("c")
        s = lax.axis_index("s")
        row_off = c * N

        for i in range(16):
            for j in range(AUGC // 16):
                zbuf[i, pl.ds(j * 16, 16)] = jnp.zeros((16,), jnp.float32)

        stripe0 = s * STRIPE
        ebase0 = s * EDGES_PER_TILE

        for r in range(2):
            def zero_body(t, _):
                pltpu.sync_copy(zbuf, acc.at[pl.ds(stripe0 + t * 16, 16)])
                return _
            lax.fori_loop(0, STRIPE // 16, zero_body, None)
            plsc.subcore_barrier()

            def chunk_body(t, _):
                base = ebase0 + t * CHUNK
                pltpu.sync_copy(edges_hbm.at[r, 0, pl.ds(base, CHUNK)], idx_v.at[0])
                pltpu.sync_copy(edges_hbm.at[r, 1, pl.ds(base, CHUNK)], idx_v.at[1])
                for i in range(CHUNK // 16):
                    idx_v[0, pl.ds(i * 16, 16)] = (
                        idx_v[0, pl.ds(i * 16, 16)] + row_off
                    )
                pltpu.async_copy(aug_hbm.at[idx_v.at[0]], rows_v, sem).wait()
                pltpu.sync_copy(rows_v, acc.at[idx_v.at[1]], add=True)
                return _
            lax.fori_loop(0, N_CHUNKS, chunk_body, None)
            plsc.subcore_barrier()

            def flush_body(t, _):
                rb = stripe0 + t * CHUNK
                pltpu.sync_copy(
                    acc.at[pl.ds(rb, CHUNK)], out_hbm.at[r, c, pl.ds(rb, CHUNK)]
                )
                return _
            lax.fori_loop(0, STRIPE // CHUNK, flush_body, None)
            plsc.subcore_barrier()

    return k(aug_flat, edges)


def _tc_combine(sums, W0, W1, bs):
    BR = 512
    grid = (ROWS_PAD // BR,)

    def body(sums_ref, w0_ref, w1_ref, b_ref, out_ref):
        sv = sums_ref[...]
        w0 = w0_ref[...]
        w1 = w1_ref[...]
        b = b_ref[...]
        out = None
        for r, w in ((0, w0), (1, w1)):
            deg = sv[r, 0, :, HALF:HALF + 1]
            inv = 1.0 / jnp.maximum(deg, 1.0)
            agg = jnp.concatenate(
                [sv[r, 0, :, :HALF], sv[r, 1, :, :HALF]], axis=1
            ) * inv
            part = lax.dot_general(
                agg, w, (((1,), (1,)), ((), ())),
                preferred_element_type=jnp.float32,
            )
            out = part if out is None else out + part
        out_ref[...] = out + b[0:1, :] + b[1:2, :]

    return pl.pallas_call(
        body,
        grid=grid,
        in_specs=[
            pl.BlockSpec((2, NC, BR, AUGC), lambda i: (0, 0, i, 0)),
            pl.BlockSpec((D, D), lambda i: (0, 0)),
            pl.BlockSpec((D, D), lambda i: (0, 0)),
            pl.BlockSpec((2, D), lambda i: (0, 0)),
        ],
        out_specs=pl.BlockSpec((BR, D), lambda i: (i, 0)),
        out_shape=jax.ShapeDtypeStruct((N, D), jnp.float32),
    )(sums, W0, W1, bs)


@jax.jit
def kernel(h, edge_index_rel0, edge_index_rel1, W_rel0, b_rel0, W_rel1, b_rel1):
    ones = jnp.ones((2, N, 1), jnp.float32)
    pad = jnp.zeros((2, N, AUGC - HALF - 1), jnp.float32)
    halves = jnp.stack([h[:, :HALF], h[:, HALF:]], axis=0)
    aug_flat = jnp.concatenate([halves, ones, pad], axis=2).reshape(2 * N, AUGC)

    def pad_edges(ei):
        src = jnp.pad(ei[0], (0, E_PAD - E))
        dst = jnp.pad(ei[1], (0, E_PAD - E), constant_values=ROWS_PAD - 1)
        return jnp.stack([src, dst], axis=0)

    edges = jnp.stack([pad_edges(edge_index_rel0), pad_edges(edge_index_rel1)], axis=0)

    sums = _sc_segment_sums(aug_flat, edges)
    bs = jnp.stack([b_rel0, b_rel1], axis=0)
    return _tc_combine(sums, W_rel0, W_rel1, bs)


# speedup vs baseline: 2.7240x; 2.7240x over previous
import functools

import jax
import jax.numpy as jnp
from jax import lax
from jax.experimental import pallas as pl
from jax.experimental.pallas import tpu as pltpu
from jax.experimental.pallas import tpu_sc as plsc

N = 10000
D = 256
E = 80000

NC = 2
NS = 16
HALF = D // 2
DEGW = 16
ROWS_PAD = 10240
EDGES_PER_TILE = 5120
E_PAD = EDGES_PER_TILE * NS
CHUNK = 128
N_CHUNKS = EDGES_PER_TILE // CHUNK
IDX_BLK = 10
STRIPE = ROWS_PAD // NS


def _sc_segment_sums(h2, edges):
    mesh = plsc.VectorSubcoreMesh(core_axis_name="c", subcore_axis_name="s")

    @functools.partial(
        pl.kernel,
        mesh=mesh,
        compiler_params=pltpu.CompilerParams(use_tc_tiling_on_sc=False),
        out_type=(
            jax.ShapeDtypeStruct((2, NC, ROWS_PAD, HALF), jnp.float32),
            jax.ShapeDtypeStruct((2, NC, ROWS_PAD, DEGW), jnp.float32),
        ),
        scratch_types=[
            pltpu.VMEM((2, IDX_BLK, CHUNK), jnp.int32),
            pltpu.VMEM((CHUNK, HALF), jnp.float32),
            pltpu.VMEM((CHUNK, DEGW), jnp.float32),
            pltpu.VMEM_SHARED((ROWS_PAD, HALF), jnp.float32),
            pltpu.VMEM_SHARED((ROWS_PAD, DEGW), jnp.float32),
            pltpu.SemaphoreType.DMA,
            pltpu.SemaphoreType.DMA,
        ],
    )
    def k(h2_hbm, edges_hbm, out_hbm, deg_hbm, eidx, rows, ones, acc, dacc,
          gs0, gs1):
        c = lax.axis_index("c")
        s = lax.axis_index("s")
        stripe0 = s * STRIPE

        for r in range(2):
            def zfill(i, _):
                for j in range(HALF // 16):
                    rows[i, pl.ds(j * 16, 16)] = jnp.zeros((16,), jnp.float32)
                ones[i] = jnp.zeros((DEGW,), jnp.float32)
                return _
            lax.fori_loop(0, CHUNK, zfill, None)

            def zero_body(t, _):
                h0 = pltpu.async_copy(
                    rows, acc.at[pl.ds(stripe0 + t * CHUNK, CHUNK)], gs0,
                )
                h1 = pltpu.async_copy(
                    ones, dacc.at[pl.ds(stripe0 + t * CHUNK, CHUNK)], gs1,
                )
                h0.wait()
                h1.wait()
                return _
            lax.fori_loop(0, STRIPE // CHUNK, zero_body, None)

            def ofill(i, _):
                ones[i] = jnp.ones((DEGW,), jnp.float32)
                return _
            lax.fori_loop(0, CHUNK, ofill, None)
            plsc.subcore_barrier()

            def blk_body(bk, _):
                for kk in range(2):
                    pltpu.sync_copy(
                        edges_hbm.at[r, kk, s, pl.ds(bk * IDX_BLK, IDX_BLK)],
                        eidx.at[kk],
                    )
                for t in range(IDX_BLK):
                    for i in range(CHUNK // 16):
                        v = eidx[0, t, pl.ds(i * 16, 16)]
                        eidx[0, t, pl.ds(i * 16, 16)] = v + v + c
                for t in range(IDX_BLK):
                    pltpu.async_copy(
                        h2_hbm.at[eidx.at[0, t]], rows, gs0
                    ).wait()
                    sh = pltpu.async_copy(
                        rows, acc.at[eidx.at[1, t]], gs0, add=True
                    )
                    dh = pltpu.async_copy(
                        ones, dacc.at[eidx.at[1, t]], gs1, add=True
                    )
                    sh.wait()
                    dh.wait()
                return _
            lax.fori_loop(0, N_CHUNKS // IDX_BLK, blk_body, None)
            plsc.subcore_barrier()

            def flush_body(t, _):
                r0 = stripe0 + t * CHUNK
                h0 = pltpu.async_copy(
                    acc.at[pl.ds(r0, CHUNK)],
                    out_hbm.at[r, c, pl.ds(r0, CHUNK)], gs0,
                )
                h1 = pltpu.async_copy(
                    dacc.at[pl.ds(r0, CHUNK)],
                    deg_hbm.at[r, c, pl.ds(r0, CHUNK)], gs1,
                )
                h0.wait()
                h1.wait()
                return _
            lax.fori_loop(0, STRIPE // CHUNK, flush_body, None)

    return k(h2, edges)


def _tc_combine(sums, degs, W0, W1, bs):
    BR = 512
    grid = (ROWS_PAD // BR,)

    def body(sums_ref, degs_ref, w0_ref, w1_ref, b_ref, out_ref):
        sv = sums_ref[...]
        dv = degs_ref[...]
        w0 = w0_ref[...]
        w1 = w1_ref[...]
        b = b_ref[...]
        out = None
        for r, w in ((0, w0), (1, w1)):
            deg = dv[r, 0, :, 0:1]
            inv = 1.0 / jnp.maximum(deg, 1.0)
            agg = jnp.concatenate([sv[r, 0], sv[r, 1]], axis=1) * inv
            part = lax.dot_general(
                agg, w, (((1,), (1,)), ((), ())),
                preferred_element_type=jnp.float32,
            )
            out = part if out is None else out + part
        out_ref[...] = out + b[0:1, :] + b[1:2, :]

    return pl.pallas_call(
        body,
        grid=grid,
        in_specs=[
            pl.BlockSpec((2, NC, BR, HALF), lambda i: (0, 0, i, 0)),
            pl.BlockSpec((2, NC, BR, DEGW), lambda i: (0, 0, i, 0)),
            pl.BlockSpec((D, D), lambda i: (0, 0)),
            pl.BlockSpec((D, D), lambda i: (0, 0)),
            pl.BlockSpec((2, D), lambda i: (0, 0)),
        ],
        out_specs=pl.BlockSpec((BR, D), lambda i: (i, 0)),
        out_shape=jax.ShapeDtypeStruct((N, D), jnp.float32),
    )(sums, degs, W0, W1, bs)


@jax.jit
def kernel(h, edge_index_rel0, edge_index_rel1, W_rel0, b_rel0, W_rel1, b_rel1):
    h2 = h.reshape(2 * N, HALF)

    def pad_edges(ei):
        src = jnp.pad(ei[0], (0, E_PAD - E)).reshape(NS, N_CHUNKS, CHUNK)
        dst = jnp.pad(
            ei[1], (0, E_PAD - E), constant_values=ROWS_PAD - 1
        ).reshape(NS, N_CHUNKS, CHUNK)
        return jnp.stack([src, dst], axis=0)

    edges = jnp.stack([pad_edges(edge_index_rel0), pad_edges(edge_index_rel1)], axis=0)

    sums, degs = _sc_segment_sums(h2, edges)
    bs = jnp.stack([b_rel0, b_rel1], axis=0)
    return _tc_combine(sums, degs, W_rel0, W_rel1, bs)


# speedup vs baseline: 2.8506x; 1.0465x over previous
import functools

import jax
import jax.numpy as jnp
from jax import lax
from jax.experimental import pallas as pl
from jax.experimental.pallas import tpu as pltpu
from jax.experimental.pallas import tpu_sc as plsc

N = 10000
D = 256
E = 80000

NC = 2
NS = 16
HALF = D // 2
DEGW = 16
ROWS_PAD = 10240
EDGES_PER_TILE = 5120
E_PAD = EDGES_PER_TILE * NS
CHUNK = 80
N_CHUNKS = EDGES_PER_TILE // CHUNK
GROUP = 16
STRIPE = ROWS_PAD // NS


def _sc_segment_sums(h2, edges):
    mesh = plsc.VectorSubcoreMesh(core_axis_name="c", subcore_axis_name="s")

    @functools.partial(
        pl.kernel,
        mesh=mesh,
        compiler_params=pltpu.CompilerParams(use_tc_tiling_on_sc=False),
        out_type=(
            jax.ShapeDtypeStruct((2, NC, ROWS_PAD, HALF), jnp.float32),
            jax.ShapeDtypeStruct((2, NC, ROWS_PAD, DEGW), jnp.float32),
        ),
        scratch_types=[
            pltpu.VMEM((2, GROUP, CHUNK), jnp.int32),
            pltpu.VMEM((2, CHUNK, HALF), jnp.float32),
            pltpu.VMEM((CHUNK, DEGW), jnp.float32),
            pltpu.VMEM_SHARED((ROWS_PAD, HALF), jnp.float32),
            pltpu.VMEM_SHARED((ROWS_PAD, DEGW), jnp.float32),
            pltpu.SemaphoreType.DMA,
            pltpu.SemaphoreType.DMA,
        ],
    )
    def k(h2_hbm, edges_hbm, out_hbm, deg_hbm, eidx, rows, ones, acc, dacc,
          gs0, gs1):
        c = lax.axis_index("c")
        s = lax.axis_index("s")
        gsem = (gs0, gs1)
        stripe0 = s * STRIPE

        for r in range(2):
            def zfill(i, _):
                for j in range(HALF // 16):
                    rows[0, i, pl.ds(j * 16, 16)] = jnp.zeros(
                        (16,), jnp.float32
                    )
                ones[i] = jnp.zeros((DEGW,), jnp.float32)
                return _
            lax.fori_loop(0, CHUNK, zfill, None)

            def zero_body(t, _):
                h0 = pltpu.async_copy(
                    rows.at[0], acc.at[pl.ds(stripe0 + t * CHUNK, CHUNK)], gs0,
                )
                h1 = pltpu.async_copy(
                    ones, dacc.at[pl.ds(stripe0 + t * CHUNK, CHUNK)], gs1,
                )
                h0.wait()
                h1.wait()
                return _
            lax.fori_loop(0, STRIPE // CHUNK, zero_body, None)

            def ofill(i, _):
                ones[i] = jnp.ones((DEGW,), jnp.float32)
                return _
            lax.fori_loop(0, CHUNK, ofill, None)
            plsc.subcore_barrier()

            def gather(t, b):
                return pltpu.async_copy(
                    h2_hbm.at[eidx.at[0, t]], rows.at[b], gsem[b]
                )

            def group_body(g, _):
                for kk in range(2):
                    pltpu.sync_copy(
                        edges_hbm.at[r, kk, s, pl.ds(g * GROUP, GROUP)],
                        eidx.at[kk],
                    )
                for t in range(GROUP):
                    for i in range(CHUNK // 16):
                        v = eidx[0, t, pl.ds(i * 16, 16)]
                        eidx[0, t, pl.ds(i * 16, 16)] = v + v + c

                cur_g = [gather(0, 0), None]
                sh = [None, None]
                dsh = [None, None]
                for j in range(GROUP):
                    b = j & 1
                    nb = 1 - b
                    cur_g[b].wait()
                    sh[b] = pltpu.async_copy(
                        rows.at[b], acc.at[eidx.at[1, j]], gsem[b], add=True
                    )
                    dsh[b] = pltpu.async_copy(
                        ones, dacc.at[eidx.at[1, j]], gsem[b], add=True
                    )
                    if j + 1 < GROUP:
                        if sh[nb] is not None:
                            sh[nb].wait()
                            dsh[nb].wait()
                        cur_g[nb] = gather(j + 1, nb)
                sh[0].wait()
                dsh[0].wait()
                sh[1].wait()
                dsh[1].wait()
                return _
            lax.fori_loop(0, N_CHUNKS // GROUP, group_body, None)
            plsc.subcore_barrier()

            def flush_body(t, _):
                r0 = stripe0 + t * CHUNK
                h0 = pltpu.async_copy(
                    acc.at[pl.ds(r0, CHUNK)],
                    out_hbm.at[r, c, pl.ds(r0, CHUNK)], gs0,
                )
                h1 = pltpu.async_copy(
                    dacc.at[pl.ds(r0, CHUNK)],
                    deg_hbm.at[r, c, pl.ds(r0, CHUNK)], gs1,
                )
                h0.wait()
                h1.wait()
                return _
            lax.fori_loop(0, STRIPE // CHUNK, flush_body, None)

    return k(h2, edges)


def _tc_combine(sums, degs, W0, W1, bs):
    BR = 512
    grid = (ROWS_PAD // BR,)

    def body(sums_ref, degs_ref, w0_ref, w1_ref, b_ref, out_ref):
        sv = sums_ref[...]
        dv = degs_ref[...]
        w0 = w0_ref[...]
        w1 = w1_ref[...]
        b = b_ref[...]
        out = None
        for r, w in ((0, w0), (1, w1)):
            deg = dv[r, 0, :, 0:1]
            inv = 1.0 / jnp.maximum(deg, 1.0)
            agg = jnp.concatenate([sv[r, 0], sv[r, 1]], axis=1) * inv
            part = lax.dot_general(
                agg, w, (((1,), (1,)), ((), ())),
                preferred_element_type=jnp.float32,
            )
            out = part if out is None else out + part
        out_ref[...] = out + b[0:1, :] + b[1:2, :]

    return pl.pallas_call(
        body,
        grid=grid,
        in_specs=[
            pl.BlockSpec((2, NC, BR, HALF), lambda i: (0, 0, i, 0)),
            pl.BlockSpec((2, NC, BR, DEGW), lambda i: (0, 0, i, 0)),
            pl.BlockSpec((D, D), lambda i: (0, 0)),
            pl.BlockSpec((D, D), lambda i: (0, 0)),
            pl.BlockSpec((2, D), lambda i: (0, 0)),
        ],
        out_specs=pl.BlockSpec((BR, D), lambda i: (i, 0)),
        out_shape=jax.ShapeDtypeStruct((N, D), jnp.float32),
    )(sums, degs, W0, W1, bs)


@jax.jit
def kernel(h, edge_index_rel0, edge_index_rel1, W_rel0, b_rel0, W_rel1, b_rel1):
    h2 = h.reshape(2 * N, HALF)

    def pad_edges(ei):
        src = jnp.pad(ei[0], (0, E_PAD - E)).reshape(NS, N_CHUNKS, CHUNK)
        dst = jnp.pad(
            ei[1], (0, E_PAD - E), constant_values=ROWS_PAD - 1
        ).reshape(NS, N_CHUNKS, CHUNK)
        return jnp.stack([src, dst], axis=0)

    edges = jnp.stack([pad_edges(edge_index_rel0), pad_edges(edge_index_rel1)], axis=0)

    sums, degs = _sc_segment_sums(h2, edges)
    bs = jnp.stack([b_rel0, b_rel1], axis=0)
    return _tc_combine(sums, degs, W_rel0, W_rel1, bs)


# speedup vs baseline: 2.9084x; 1.0203x over previous
import functools

import jax
import jax.numpy as jnp
from jax import lax
from jax.experimental import pallas as pl
from jax.experimental.pallas import tpu as pltpu
from jax.experimental.pallas import tpu_sc as plsc

N = 10000
D = 256
E = 80000

NC = 2
NS = 16
HALF = D // 2
DEGW = 16
ROWS_PAD = 10240
EDGES_PER_TILE = 5120
E_PAD = EDGES_PER_TILE * NS
CHUNK = 80
N_CHUNKS = EDGES_PER_TILE // CHUNK
GROUP = 32
STRIPE = ROWS_PAD // NS


def _sc_segment_sums(h2, edges):
    mesh = plsc.VectorSubcoreMesh(core_axis_name="c", subcore_axis_name="s")

    @functools.partial(
        pl.kernel,
        mesh=mesh,
        compiler_params=pltpu.CompilerParams(use_tc_tiling_on_sc=False),
        out_type=(
            jax.ShapeDtypeStruct((2, NC, ROWS_PAD, HALF), jnp.float32),
            jax.ShapeDtypeStruct((2, NC, ROWS_PAD, DEGW), jnp.float32),
        ),
        scratch_types=[
            pltpu.VMEM((2, GROUP, CHUNK), jnp.int32),
            pltpu.VMEM((2, CHUNK, HALF), jnp.float32),
            pltpu.VMEM((CHUNK, DEGW), jnp.float32),
            pltpu.VMEM_SHARED((ROWS_PAD, HALF), jnp.float32),
            pltpu.VMEM_SHARED((ROWS_PAD, DEGW), jnp.float32),
            pltpu.SemaphoreType.DMA,
            pltpu.SemaphoreType.DMA,
        ],
    )
    def k(h2_hbm, edges_hbm, out_hbm, deg_hbm, eidx, rows, ones, acc, dacc,
          gs0, gs1):
        c = lax.axis_index("c")
        s = lax.axis_index("s")
        gsem = (gs0, gs1)
        stripe0 = s * STRIPE

        for r in range(2):
            def zfill(i, _):
                for j in range(HALF // 16):
                    rows[0, i, pl.ds(j * 16, 16)] = jnp.zeros(
                        (16,), jnp.float32
                    )
                ones[i] = jnp.zeros((DEGW,), jnp.float32)
                return _
            lax.fori_loop(0, CHUNK, zfill, None)

            def zero_body(t, _):
                h0 = pltpu.async_copy(
                    rows.at[0], acc.at[pl.ds(stripe0 + t * CHUNK, CHUNK)], gs0,
                )
                h1 = pltpu.async_copy(
                    ones, dacc.at[pl.ds(stripe0 + t * CHUNK, CHUNK)], gs1,
                )
                h0.wait()
                h1.wait()
                return _
            lax.fori_loop(0, STRIPE // CHUNK, zero_body, None)

            def ofill(i, _):
                ones[i] = jnp.ones((DEGW,), jnp.float32)
                return _
            lax.fori_loop(0, CHUNK, ofill, None)
            plsc.subcore_barrier()

            def gather(t, b):
                return pltpu.async_copy(
                    h2_hbm.at[eidx.at[0, t]], rows.at[b], gsem[b]
                )

            def group_body(g, _):
                for kk in range(2):
                    pltpu.sync_copy(
                        edges_hbm.at[r, kk, s, pl.ds(g * GROUP, GROUP)],
                        eidx.at[kk],
                    )
                for t in range(GROUP):
                    for i in range(CHUNK // 16):
                        v = eidx[0, t, pl.ds(i * 16, 16)]
                        eidx[0, t, pl.ds(i * 16, 16)] = v + v + c

                cur_g = [gather(0, 0), None]
                sh = [None, None]
                dsh = [None, None]
                for j in range(GROUP):
                    b = j & 1
                    nb = 1 - b
                    cur_g[b].wait()
                    sh[b] = pltpu.async_copy(
                        rows.at[b], acc.at[eidx.at[1, j]], gsem[b], add=True
                    )
                    dsh[b] = pltpu.async_copy(
                        ones, dacc.at[eidx.at[1, j]], gsem[b], add=True
                    )
                    if j + 1 < GROUP:
                        if sh[nb] is not None:
                            sh[nb].wait()
                            dsh[nb].wait()
                        cur_g[nb] = gather(j + 1, nb)
                sh[0].wait()
                dsh[0].wait()
                sh[1].wait()
                dsh[1].wait()
                return _
            lax.fori_loop(0, N_CHUNKS // GROUP, group_body, None)
            plsc.subcore_barrier()

            def flush_body(t, _):
                r0 = stripe0 + t * CHUNK
                h0 = pltpu.async_copy(
                    acc.at[pl.ds(r0, CHUNK)],
                    out_hbm.at[r, c, pl.ds(r0, CHUNK)], gs0,
                )
                h1 = pltpu.async_copy(
                    dacc.at[pl.ds(r0, CHUNK)],
                    deg_hbm.at[r, c, pl.ds(r0, CHUNK)], gs1,
                )
                h0.wait()
                h1.wait()
                return _
            lax.fori_loop(0, STRIPE // CHUNK, flush_body, None)

    return k(h2, edges)


def _tc_combine(sums, degs, W0, W1, bs):
    BR = 512
    grid = (ROWS_PAD // BR,)

    def body(sums_ref, degs_ref, w0_ref, w1_ref, b_ref, out_ref):
        sv = sums_ref[...]
        dv = degs_ref[...]
        w0 = w0_ref[...]
        w1 = w1_ref[...]
        b = b_ref[...]
        out = None
        for r, w in ((0, w0), (1, w1)):
            deg = dv[r, 0, :, 0:1]
            inv = 1.0 / jnp.maximum(deg, 1.0)
            agg = jnp.concatenate([sv[r, 0], sv[r, 1]], axis=1) * inv
            part = lax.dot_general(
                agg, w, (((1,), (1,)), ((), ())),
                preferred_element_type=jnp.float32,
            )
            out = part if out is None else out + part
        out_ref[...] = out + b[0:1, :] + b[1:2, :]

    return pl.pallas_call(
        body,
        grid=grid,
        in_specs=[
            pl.BlockSpec((2, NC, BR, HALF), lambda i: (0, 0, i, 0)),
            pl.BlockSpec((2, NC, BR, DEGW), lambda i: (0, 0, i, 0)),
            pl.BlockSpec((D, D), lambda i: (0, 0)),
            pl.BlockSpec((D, D), lambda i: (0, 0)),
            pl.BlockSpec((2, D), lambda i: (0, 0)),
        ],
        out_specs=pl.BlockSpec((BR, D), lambda i: (i, 0)),
        out_shape=jax.ShapeDtypeStruct((N, D), jnp.float32),
    )(sums, degs, W0, W1, bs)


@jax.jit
def kernel(h, edge_index_rel0, edge_index_rel1, W_rel0, b_rel0, W_rel1, b_rel1):
    h2 = h.reshape(2 * N, HALF)

    def pad_edges(ei):
        src = jnp.pad(ei[0], (0, E_PAD - E)).reshape(NS, N_CHUNKS, CHUNK)
        dst = jnp.pad(
            ei[1], (0, E_PAD - E), constant_values=ROWS_PAD - 1
        ).reshape(NS, N_CHUNKS, CHUNK)
        return jnp.stack([src, dst], axis=0)

    edges = jnp.stack([pad_edges(edge_index_rel0), pad_edges(edge_index_rel1)], axis=0)

    sums, degs = _sc_segment_sums(h2, edges)
    bs = jnp.stack([b_rel0, b_rel1], axis=0)
    return _tc_combine(sums, degs, W_rel0, W_rel1, bs)
